# baseline (speedup 1.0000x reference)
import functools

import jax
import jax.numpy as jnp
from jax import lax
from jax.experimental import pallas as pl
from jax.experimental.pallas import tpu as pltpu
from jax.experimental.pallas import tpu_sc as plsc

_NC, _NS, _L = 2, 16, 16
_NW = _NC * _NS
_ROWS = 16


def kernel(x, pos_table):
    batch, seq, hidden = x.shape
    pos = pos_table[:seq]
    sb_total = seq // _ROWS
    sb_per_w = sb_total // _NW
    chunks = hidden // _L

    mesh = plsc.VectorSubcoreMesh(core_axis_name="c", subcore_axis_name="s")

    @functools.partial(
        pl.kernel,
        mesh=mesh,
        out_type=jax.ShapeDtypeStruct(x.shape, x.dtype),
        scratch_types=[
            pltpu.VMEM((2, _ROWS, hidden), x.dtype),
            pltpu.VMEM((_ROWS, hidden), x.dtype),
            pltpu.SemaphoreType.DMA((2,)),
            pltpu.SemaphoreType.DMA((2,)),
        ],
    )
    def run(x_hbm, pos_hbm, o_hbm, xbuf, posbuf, lsem, ssem):
        wid = lax.axis_index("s") * _NC + lax.axis_index("c")

        def add_pos(buf):
            @pl.loop(0, _ROWS)
            def _(r):
                for j in range(chunks):
                    v = posbuf.at[r, pl.ds(j * _L, _L)][...]
                    plsc.addupdate(buf.at[r, pl.ds(j * _L, _L)], v)

        @pl.loop(0, sb_per_w)
        def _(si):
            s0 = (wid * sb_per_w + si) * _ROWS
            pltpu.sync_copy(pos_hbm.at[pl.ds(s0, _ROWS), :], posbuf)
            loads = []
            stores = []
            loads.append(
                pltpu.async_copy(
                    x_hbm.at[0, pl.ds(s0, _ROWS), :], xbuf.at[0], lsem.at[0]
                )
            )
            for b in range(batch):
                cur = b % 2
                if b + 1 < batch:
                    if b >= 1:
                        stores[b - 1].wait()
                    loads.append(
                        pltpu.async_copy(
                            x_hbm.at[b + 1, pl.ds(s0, _ROWS), :],
                            xbuf.at[1 - cur],
                            lsem.at[1 - cur],
                        )
                    )
                loads[b].wait()
                add_pos(xbuf.at[cur])
                stores.append(
                    pltpu.async_copy(
                        xbuf.at[cur],
                        o_hbm.at[b, pl.ds(s0, _ROWS), :],
                        ssem.at[cur],
                    )
                )
            stores[batch - 2].wait()
            stores[batch - 1].wait()

    return run(x, pos)


# speedup vs baseline: 1.1256x; 1.1256x over previous
import functools

import jax
import jax.numpy as jnp
from jax import lax
from jax.experimental import pallas as pl
from jax.experimental.pallas import tpu as pltpu
from jax.experimental.pallas import tpu_sc as plsc

_NC, _NS, _L = 2, 16, 16
_NW = _NC * _NS
_ROWS = 16


def kernel(x, pos_table):
    batch, seq, hidden = x.shape
    pos = pos_table[:seq]
    sb_total = seq // _ROWS
    sb_per_w = sb_total // _NW
    chunks = hidden // _L

    mesh = plsc.VectorSubcoreMesh(core_axis_name="c", subcore_axis_name="s")

    @functools.partial(
        pl.kernel,
        mesh=mesh,
        out_type=jax.ShapeDtypeStruct(x.shape, x.dtype),
        scratch_types=[
            pltpu.VMEM((2, _ROWS, hidden), x.dtype),
            pltpu.VMEM((_ROWS, hidden), x.dtype),
            pltpu.SemaphoreType.DMA((2,)),
            pltpu.SemaphoreType.DMA((2,)),
        ],
    )
    def run(x_hbm, pos_hbm, o_hbm, xbuf, posbuf, lsem, ssem):
        wid = lax.axis_index("s") * _NC + lax.axis_index("c")

        def add_pos(buf):
            @plsc.parallel_loop(0, _ROWS, unroll=4)
            def _(r):
                for j in range(chunks):
                    v = posbuf.at[r, pl.ds(j * _L, _L)][...]
                    plsc.addupdate(buf.at[r, pl.ds(j * _L, _L)], v)

        @pl.loop(0, sb_per_w)
        def _(si):
            s0 = (wid * sb_per_w + si) * _ROWS
            pltpu.sync_copy(pos_hbm.at[pl.ds(s0, _ROWS), :], posbuf)
            loads = []
            stores = []
            loads.append(
                pltpu.async_copy(
                    x_hbm.at[0, pl.ds(s0, _ROWS), :], xbuf.at[0], lsem.at[0]
                )
            )
            for b in range(batch):
                cur = b % 2
                if b + 1 < batch:
                    if b >= 1:
                        stores[b - 1].wait()
                    loads.append(
                        pltpu.async_copy(
                            x_hbm.at[b + 1, pl.ds(s0, _ROWS), :],
                            xbuf.at[1 - cur],
                            lsem.at[1 - cur],
                        )
                    )
                loads[b].wait()
                add_pos(xbuf.at[cur])
                stores.append(
                    pltpu.async_copy(
                        xbuf.at[cur],
                        o_hbm.at[b, pl.ds(s0, _ROWS), :],
                        ssem.at[cur],
                    )
                )
            stores[batch - 2].wait()
            stores[batch - 1].wait()

    return run(x, pos)


# speedup vs baseline: 3.6031x; 3.2011x over previous
import jax
import jax.numpy as jnp
from jax.experimental import pallas as pl
from jax.experimental.pallas import tpu as pltpu

_SEQ_BLOCK = 1024


def _add_kernel(x_ref, pos_ref, o_ref):
    o_ref[...] = x_ref[...] + pos_ref[...]


def kernel(x, pos_table):
    batch, seq_len, hidden = x.shape
    pos = pos_table[:seq_len]
    sblocks = seq_len // _SEQ_BLOCK

    grid = (sblocks, batch // 2)
    out = pl.pallas_call(
        _add_kernel,
        grid=grid,
        in_specs=[
            pl.BlockSpec((2, _SEQ_BLOCK, hidden), lambda s, b: (b, s, 0)),
            pl.BlockSpec((_SEQ_BLOCK, hidden), lambda s, b: (s, 0)),
        ],
        out_specs=pl.BlockSpec((2, _SEQ_BLOCK, hidden), lambda s, b: (b, s, 0)),
        out_shape=jax.ShapeDtypeStruct((batch, seq_len, hidden), x.dtype),
        compiler_params=pltpu.CompilerParams(
            dimension_semantics=("arbitrary", "arbitrary"),
        ),
    )(x, pos)
    return out
